# baseline (speedup 1.0000x reference)
import functools

import jax
import jax.numpy as jnp
from jax import lax
from jax.experimental import pallas as pl
from jax.experimental.pallas import tpu as pltpu
from jax.experimental.pallas import tpu_sc as plsc

_K = 4
_NC = 1 << _K
_SPLIT = 3840


def _moebius_js():
    jj = lax.broadcasted_iota(jnp.int32, (_NC, _NC), 0)
    ss = lax.broadcasted_iota(jnp.int32, (_NC, _NC), 1)
    subset = (jj & ss) == jj
    d = ss ^ jj
    pc = (d & 1) + ((d >> 1) & 1) + ((d >> 2) & 1) + ((d >> 3) & 1)
    sign = (1 - 2 * (pc & 1)).astype(jnp.float32)
    return jnp.where(subset, sign, 0.0)


def _make_sc_gather(b, tk):
    info = plsc.get_sparse_core_info()
    ncores, nsub = info.num_cores, info.num_subcores
    nw = ncores * nsub
    rows_per_w = tk // nw
    ch = min(rows_per_w, 8)
    nch = rows_per_w // ch
    mesh = plsc.VectorSubcoreMesh(core_axis_name="c", subcore_axis_name="s")

    @functools.partial(
        pl.kernel,
        out_type=jax.ShapeDtypeStruct((tk, b), jnp.float32),
        mesh=mesh,
        scratch_types=[
            pltpu.VMEM((ch,), jnp.int32),
            pltpu.VMEM((ch,), jnp.int32),
            pltpu.VMEM((ch, b), jnp.float32),
            pltpu.VMEM((ch, b), jnp.float32),
            pltpu.SemaphoreType.DMA,
            pltpu.SemaphoreType.DMA,
            pltpu.SemaphoreType.DMA,
            pltpu.SemaphoreType.DMA,
        ],
    )
    def gather(xt_hbm, idx_hbm, out_hbm, i0, i1, r0, r1, g0, g1, w0, w1):
        wid = lax.axis_index("s") * ncores + lax.axis_index("c")
        base = wid * rows_per_w
        idx = (i0, i1)
        rows = (r0, r1)
        gsem = (g0, g1)
        wsem = (w0, w1)

        def out_at(c):
            return out_hbm.at[pl.ds(base + c * ch, ch)]

        for c in range(nch):
            s = c % 2
            if c >= 2:
                pltpu.make_async_copy(rows[s], out_at(c - 2), wsem[s]).wait()
            pltpu.sync_copy(idx_hbm.at[pl.ds(base + c * ch, ch)], idx[s])
            pltpu.async_copy(xt_hbm.at[idx[s]], rows[s], gsem[s])
            if c >= 1:
                p = (c - 1) % 2
                pltpu.make_async_copy(xt_hbm.at[idx[p]], rows[p], gsem[p]).wait()
                pltpu.async_copy(rows[p], out_at(c - 1), wsem[p])
        last = nch - 1
        s = last % 2
        pltpu.make_async_copy(xt_hbm.at[idx[s]], rows[s], gsem[s]).wait()
        pltpu.async_copy(rows[s], out_at(last), wsem[s])
        if nch >= 2:
            p = (last - 1) % 2
            pltpu.make_async_copy(rows[p], out_at(last - 1), wsem[p]).wait()
        pltpu.make_async_copy(rows[s], out_at(last), wsem[s]).wait()

    return gather


def _dense_body(g_ref, cpd_ref, out_ref):
    t = g_ref.shape[0] // _K

    g0 = g_ref[0 * t:1 * t, :]
    g1 = g_ref[1 * t:2 * t, :]
    g2 = g_ref[2 * t:3 * t, :]
    g3 = g_ref[3 * t:4 * t, :]

    cp = jnp.dot(cpd_ref[...], _moebius_js(),
                 preferred_element_type=jnp.float32)

    q3 = g2 * g3
    p3 = g0 * g1

    def r(i):
        return (cp[:, 4 * i + 0:4 * i + 1]
                + cp[:, 4 * i + 1:4 * i + 2] * g3
                + cp[:, 4 * i + 2:4 * i + 3] * g2
                + cp[:, 4 * i + 3:4 * i + 4] * q3)

    inner = r(0) + r(1) * g1 + r(2) * g0 + r(3) * p3
    out_ref[...] = jnp.sum(inner, axis=0, keepdims=True)


def _dense_tc(g, cpd, b, t, tk):
    bb = min(b, 512)
    out = pl.pallas_call(
        _dense_body,
        grid=(b // bb,),
        in_specs=[
            pl.BlockSpec((tk, bb), lambda i: (0, i)),
            pl.BlockSpec((t, _NC), lambda i: (0, 0)),
        ],
        out_specs=pl.BlockSpec((1, bb), lambda i: (0, i)),
        out_shape=jax.ShapeDtypeStruct((1, b), jnp.float32),
    )(g, cpd)
    return out


def _fused_body(x_ref, fvt_ref, cpdt_ref, out_ref, sel_ref):
    bb, v = x_ref.shape
    tk = fvt_ref.shape[1]
    t = tk // _K

    @pl.when(pl.program_id(0) == 0)
    def _():
        iota_v = lax.broadcasted_iota(jnp.int32, (v, tk), 0)
        sel_ref[...] = (iota_v == fvt_ref[...]).astype(jnp.float32)

    g = jnp.dot(x_ref[...], sel_ref[...], preferred_element_type=jnp.float32)

    g0 = g[:, 0 * t:1 * t]
    g1 = g[:, 1 * t:2 * t]
    g2 = g[:, 2 * t:3 * t]
    g3 = g[:, 3 * t:4 * t]

    cp = jnp.dot(_moebius_js().T, cpdt_ref[...],
                 preferred_element_type=jnp.float32)

    q3 = g2 * g3
    p3 = g0 * g1

    def r(i):
        return (cp[4 * i + 0][None, :]
                + cp[4 * i + 1][None, :] * g3
                + cp[4 * i + 2][None, :] * g2
                + cp[4 * i + 3][None, :] * q3)

    inner = r(0) + r(1) * g1 + r(2) * g0 + r(3) * p3
    out_ref[...] = jnp.sum(inner, axis=1)[None, :]


def _fused_tc(x, fvt, cpdt, bs, v, t, tk):
    bb = min(bs, 768)
    out = pl.pallas_call(
        _fused_body,
        grid=(bs // bb,),
        in_specs=[
            pl.BlockSpec((bb, v), lambda i: (i, 0)),
            pl.BlockSpec((1, tk), lambda i: (0, 0)),
            pl.BlockSpec((_NC, t), lambda i: (0, 0)),
        ],
        out_specs=pl.BlockSpec((1, bb), lambda i: (0, i)),
        out_shape=jax.ShapeDtypeStruct((1, bs), jnp.float32),
        scratch_shapes=[pltpu.VMEM((v, tk), jnp.float32)],
    )(x, fvt, cpdt)
    return out


def kernel(x, func_vars, cpd):
    b, v = x.shape
    t, k = func_vars.shape
    assert k == _K
    tk = t * k
    fvt = func_vars.T.reshape(1, tk).astype(jnp.int32)
    fv_flat = fvt.reshape(tk)

    bs = _SPLIT
    b1 = b - bs

    xt1 = x[bs:].T
    g1 = _make_sc_gather(b1, tk)(xt1, fv_flat)
    out0 = _fused_tc(x, fvt, cpd.T, bs, v, t, tk)
    out1 = _dense_tc(g1, cpd, b1, t, tk)
    return jnp.concatenate([out0, out1], axis=1).reshape(b)


# speedup vs baseline: 1.0318x; 1.0318x over previous
import functools

import jax
import jax.numpy as jnp
from jax import lax
from jax.experimental import pallas as pl
from jax.experimental.pallas import tpu as pltpu
from jax.experimental.pallas import tpu_sc as plsc

_K = 4
_NC = 1 << _K
_SPLIT = 3584


def _moebius_js():
    jj = lax.broadcasted_iota(jnp.int32, (_NC, _NC), 0)
    ss = lax.broadcasted_iota(jnp.int32, (_NC, _NC), 1)
    subset = (jj & ss) == jj
    d = ss ^ jj
    pc = (d & 1) + ((d >> 1) & 1) + ((d >> 2) & 1) + ((d >> 3) & 1)
    sign = (1 - 2 * (pc & 1)).astype(jnp.float32)
    return jnp.where(subset, sign, 0.0)


def _make_sc_gather(b, tk):
    info = plsc.get_sparse_core_info()
    ncores, nsub = info.num_cores, info.num_subcores
    nw = ncores * nsub
    rows_per_w = tk // nw
    ch = min(rows_per_w, 8)
    nch = rows_per_w // ch
    mesh = plsc.VectorSubcoreMesh(core_axis_name="c", subcore_axis_name="s")

    @functools.partial(
        pl.kernel,
        out_type=jax.ShapeDtypeStruct((tk, b), jnp.float32),
        mesh=mesh,
        scratch_types=[
            pltpu.VMEM((ch,), jnp.int32),
            pltpu.VMEM((ch,), jnp.int32),
            pltpu.VMEM((ch, b), jnp.float32),
            pltpu.VMEM((ch, b), jnp.float32),
            pltpu.SemaphoreType.DMA,
            pltpu.SemaphoreType.DMA,
            pltpu.SemaphoreType.DMA,
            pltpu.SemaphoreType.DMA,
        ],
    )
    def gather(xt_hbm, idx_hbm, out_hbm, i0, i1, r0, r1, g0, g1, w0, w1):
        wid = lax.axis_index("s") * ncores + lax.axis_index("c")
        base = wid * rows_per_w
        idx = (i0, i1)
        rows = (r0, r1)
        gsem = (g0, g1)
        wsem = (w0, w1)

        def out_at(c):
            return out_hbm.at[pl.ds(base + c * ch, ch)]

        for c in range(nch):
            s = c % 2
            if c >= 2:
                pltpu.make_async_copy(rows[s], out_at(c - 2), wsem[s]).wait()
            pltpu.sync_copy(idx_hbm.at[pl.ds(base + c * ch, ch)], idx[s])
            pltpu.async_copy(xt_hbm.at[idx[s]], rows[s], gsem[s])
            if c >= 1:
                p = (c - 1) % 2
                pltpu.make_async_copy(xt_hbm.at[idx[p]], rows[p], gsem[p]).wait()
                pltpu.async_copy(rows[p], out_at(c - 1), wsem[p])
        last = nch - 1
        s = last % 2
        pltpu.make_async_copy(xt_hbm.at[idx[s]], rows[s], gsem[s]).wait()
        pltpu.async_copy(rows[s], out_at(last), wsem[s])
        if nch >= 2:
            p = (last - 1) % 2
            pltpu.make_async_copy(rows[p], out_at(last - 1), wsem[p]).wait()
        pltpu.make_async_copy(rows[s], out_at(last), wsem[s]).wait()

    return gather


def _dense_body(g_ref, cpd_ref, out_ref):
    t = g_ref.shape[0] // _K

    g0 = g_ref[0 * t:1 * t, :]
    g1 = g_ref[1 * t:2 * t, :]
    g2 = g_ref[2 * t:3 * t, :]
    g3 = g_ref[3 * t:4 * t, :]

    cp = jnp.dot(cpd_ref[...], _moebius_js(),
                 preferred_element_type=jnp.float32)

    q3 = g2 * g3
    p3 = g0 * g1

    def r(i):
        return (cp[:, 4 * i + 0:4 * i + 1]
                + cp[:, 4 * i + 1:4 * i + 2] * g3
                + cp[:, 4 * i + 2:4 * i + 3] * g2
                + cp[:, 4 * i + 3:4 * i + 4] * q3)

    inner = r(0) + r(1) * g1 + r(2) * g0 + r(3) * p3
    out_ref[...] = jnp.sum(inner, axis=0, keepdims=True)


def _dense_tc(g, cpd, b, t, tk):
    bb = min(b, 512)
    out = pl.pallas_call(
        _dense_body,
        grid=(b // bb,),
        in_specs=[
            pl.BlockSpec((tk, bb), lambda i: (0, i)),
            pl.BlockSpec((t, _NC), lambda i: (0, 0)),
        ],
        out_specs=pl.BlockSpec((1, bb), lambda i: (0, i)),
        out_shape=jax.ShapeDtypeStruct((1, b), jnp.float32),
    )(g, cpd)
    return out


def _fused_body(x_ref, fvt_ref, cpdt_ref, out_ref, sel_ref):
    bb, v = x_ref.shape
    tk = fvt_ref.shape[1]
    t = tk // _K

    @pl.when(pl.program_id(0) == 0)
    def _():
        iota_v = lax.broadcasted_iota(jnp.int32, (v, tk), 0)
        sel_ref[...] = (iota_v == fvt_ref[...]).astype(jnp.float32)

    g = jnp.dot(x_ref[...], sel_ref[...], preferred_element_type=jnp.float32)

    g0 = g[:, 0 * t:1 * t]
    g1 = g[:, 1 * t:2 * t]
    g2 = g[:, 2 * t:3 * t]
    g3 = g[:, 3 * t:4 * t]

    cp = jnp.dot(_moebius_js().T, cpdt_ref[...],
                 preferred_element_type=jnp.float32)

    q3 = g2 * g3
    p3 = g0 * g1

    def r(i):
        return (cp[4 * i + 0][None, :]
                + cp[4 * i + 1][None, :] * g3
                + cp[4 * i + 2][None, :] * g2
                + cp[4 * i + 3][None, :] * q3)

    inner = r(0) + r(1) * g1 + r(2) * g0 + r(3) * p3
    out_ref[...] = jnp.sum(inner, axis=1)[None, :]


def _fused_tc(x, fvt, cpdt, bs, v, t, tk):
    bb = min(bs, 896)
    out = pl.pallas_call(
        _fused_body,
        grid=(bs // bb,),
        in_specs=[
            pl.BlockSpec((bb, v), lambda i: (i, 0)),
            pl.BlockSpec((1, tk), lambda i: (0, 0)),
            pl.BlockSpec((_NC, t), lambda i: (0, 0)),
        ],
        out_specs=pl.BlockSpec((1, bb), lambda i: (0, i)),
        out_shape=jax.ShapeDtypeStruct((1, bs), jnp.float32),
        scratch_shapes=[pltpu.VMEM((v, tk), jnp.float32)],
    )(x, fvt, cpdt)
    return out


def kernel(x, func_vars, cpd):
    b, v = x.shape
    t, k = func_vars.shape
    assert k == _K
    tk = t * k
    fvt = func_vars.T.reshape(1, tk).astype(jnp.int32)
    fv_flat = fvt.reshape(tk)

    bs = _SPLIT
    b1 = b - bs

    xt1 = x[bs:].T
    g1 = _make_sc_gather(b1, tk)(xt1, fv_flat)
    out0 = _fused_tc(x, fvt, cpd.T, bs, v, t, tk)
    out1 = _dense_tc(g1, cpd, b1, t, tk)
    return jnp.concatenate([out0, out1], axis=1).reshape(b)
